# baseline (speedup 1.0000x reference)
import jax
import jax.numpy as jnp
from jax.experimental import pallas as pl
from jax.experimental.pallas import tpu as pltpu


_BLOCK_N = 5000


def _dot(a, b):
    return jnp.dot(a.astype(jnp.bfloat16), b.astype(jnp.bfloat16),
                   preferred_element_type=jnp.float32)


def _fused_mlp_kernel(x_ref, w_in_ref, w_h1_ref, w_h2_ref, w_out_ref,
                      w_cls_t_ref, b_cls_ref, out_ref, h_scr_ref):
    n_chunks = pl.num_programs(1)
    chunk = x_ref.shape[1]
    j = pl.program_id(1)
    x = x_ref[0]
    h = jax.nn.relu(_dot(x, w_in_ref[...]).astype(jnp.bfloat16))
    h = jax.nn.relu(_dot(h, w_h1_ref[...]).astype(jnp.bfloat16))
    h = jax.nn.relu(_dot(h, w_h2_ref[...]).astype(jnp.bfloat16))
    h = _dot(h, w_out_ref[...]).astype(jnp.bfloat16)
    h_scr_ref[pl.ds(j * chunk, chunk), :] = h

    @pl.when(j == n_chunks - 1)
    def _classify():
        y_t = jax.lax.dot_general(
            w_cls_t_ref[...].astype(jnp.bfloat16), h_scr_ref[...],
            dimension_numbers=(((1,), (1,)), ((), ())),
            preferred_element_type=jnp.float32)
        b = jax.lax.broadcast_in_dim(b_cls_ref[0], y_t.shape, (0,))
        out_ref[0] = y_t + b


def kernel(h_0, W_in, W_h1, W_h2, W_out, W_cls, b_cls):
    B, N, D_in = h_0.shape
    D_h = W_in.shape[1]
    D_out = W_cls.shape[1]
    W_cls_t = W_cls.T
    b2 = b_cls.reshape(1, D_out)

    chunk = _BLOCK_N if N % _BLOCK_N == 0 else N
    grid = (B, N // chunk)

    def w_spec(shape):
        return pl.BlockSpec(shape, lambda b, i: (0, 0))

    y_t = pl.pallas_call(
        _fused_mlp_kernel,
        grid=grid,
        in_specs=[
            pl.BlockSpec((1, chunk, D_in), lambda b, i: (b, i, 0)),
            w_spec((D_in, D_h)),
            w_spec((D_h, D_h)),
            w_spec((D_h, D_h)),
            w_spec((D_h, D_h)),
            w_spec((D_out, D_h)),
            w_spec((1, D_out)),
        ],
        out_specs=pl.BlockSpec((1, D_out, N), lambda b, i: (b, 0, 0)),
        out_shape=jax.ShapeDtypeStruct((B, D_out, N), jnp.float32),
        scratch_shapes=[pltpu.VMEM((N, D_h), jnp.bfloat16)],
        compiler_params=pltpu.CompilerParams(
            dimension_semantics=("parallel", "arbitrary")),
    )(h_0, W_in, W_h1, W_h2, W_out, W_cls_t, b2)

    return jnp.transpose(y_t, (0, 2, 1))


# speedup vs baseline: 1.0660x; 1.0660x over previous
import jax
import jax.numpy as jnp
from jax.experimental import pallas as pl
from jax.experimental.pallas import tpu as pltpu


_BLOCK_N = 5000


def _dot(a, b):
    return jnp.dot(a.astype(jnp.bfloat16), b.astype(jnp.bfloat16),
                   preferred_element_type=jnp.float32)


_N_CHUNKS = 2


def _fused_mlp_kernel(x_ref, w_in_ref, w_h1_ref, w_h2_ref, w_out_ref,
                      w_cls_t_ref, b_cls_ref, out_ref):
    n = x_ref.shape[1]
    chunk = n // _N_CHUNKS
    hs = []
    for c in range(_N_CHUNKS):
        x = x_ref[0, c * chunk:(c + 1) * chunk, :]
        h = jax.nn.relu(_dot(x, w_in_ref[...]).astype(jnp.bfloat16))
        h = jax.nn.relu(_dot(h, w_h1_ref[...]).astype(jnp.bfloat16))
        h = jax.nn.relu(_dot(h, w_h2_ref[...]).astype(jnp.bfloat16))
        hs.append(_dot(h, w_out_ref[...]).astype(jnp.bfloat16))
    h_full = jnp.concatenate(hs, axis=0) if _N_CHUNKS > 1 else hs[0]
    y_t = jax.lax.dot_general(
        w_cls_t_ref[...].astype(jnp.bfloat16), h_full,
        dimension_numbers=(((1,), (1,)), ((), ())),
        preferred_element_type=jnp.float32)
    b = jax.lax.broadcast_in_dim(b_cls_ref[0], y_t.shape, (0,))
    out_ref[0] = y_t + b


def kernel(h_0, W_in, W_h1, W_h2, W_out, W_cls, b_cls):
    B, N, D_in = h_0.shape
    D_h = W_in.shape[1]
    D_out = W_cls.shape[1]
    W_cls_t = W_cls.T
    b2 = b_cls.reshape(1, D_out)

    grid = (B,)

    def w_spec(shape):
        return pl.BlockSpec(shape, lambda b: (0, 0))

    y_t = pl.pallas_call(
        _fused_mlp_kernel,
        grid=grid,
        in_specs=[
            pl.BlockSpec((1, N, D_in), lambda b: (b, 0, 0)),
            w_spec((D_in, D_h)),
            w_spec((D_h, D_h)),
            w_spec((D_h, D_h)),
            w_spec((D_h, D_h)),
            w_spec((D_out, D_h)),
            w_spec((1, D_out)),
        ],
        out_specs=pl.BlockSpec((1, D_out, N), lambda b: (b, 0, 0)),
        out_shape=jax.ShapeDtypeStruct((B, D_out, N), jnp.float32),
        compiler_params=pltpu.CompilerParams(
            dimension_semantics=("parallel",)),
    )(h_0, W_in, W_h1, W_h2, W_out, W_cls_t, b2)

    return jnp.transpose(y_t, (0, 2, 1))


# speedup vs baseline: 1.2310x; 1.1548x over previous
import jax
import jax.numpy as jnp
from jax.experimental import pallas as pl
from jax.experimental.pallas import tpu as pltpu


_BLOCK_N = 5000


def _dot(a, b):
    return jnp.dot(a.astype(jnp.bfloat16), b.astype(jnp.bfloat16),
                   preferred_element_type=jnp.float32)


_N_CHUNKS = 4


def _fused_mlp_kernel(x_ref, w_in_ref, w_h1_ref, w_h2_ref, w_out_ref,
                      w_cls_t_ref, b_cls_ref, out_ref):
    x = x_ref[0]
    h = jax.nn.relu(_dot(x, w_in_ref[...]).astype(jnp.bfloat16))
    h = jax.nn.relu(_dot(h, w_h1_ref[...]).astype(jnp.bfloat16))
    h = jax.nn.relu(_dot(h, w_h2_ref[...]).astype(jnp.bfloat16))
    w_comb_t = jax.lax.dot_general(
        w_cls_t_ref[...].astype(jnp.bfloat16),
        w_out_ref[...].astype(jnp.bfloat16),
        dimension_numbers=(((1,), (1,)), ((), ())),
        preferred_element_type=jnp.float32)
    y_t = jax.lax.dot_general(
        w_comb_t.astype(jnp.bfloat16), h,
        dimension_numbers=(((1,), (1,)), ((), ())),
        preferred_element_type=jnp.float32)
    b = jax.lax.broadcast_in_dim(b_cls_ref[0], y_t.shape, (0,))
    out_ref[0] = y_t + b


def kernel(h_0, W_in, W_h1, W_h2, W_out, W_cls, b_cls):
    B, N, D_in = h_0.shape
    D_h = W_in.shape[1]
    D_out = W_cls.shape[1]
    W_cls_t = W_cls.T
    b2 = b_cls.reshape(1, D_out)

    grid = (B,)

    def w_spec(shape):
        return pl.BlockSpec(shape, lambda b: (0, 0))

    y_t = pl.pallas_call(
        _fused_mlp_kernel,
        grid=grid,
        in_specs=[
            pl.BlockSpec((1, N, D_in), lambda b: (b, 0, 0)),
            w_spec((D_in, D_h)),
            w_spec((D_h, D_h)),
            w_spec((D_h, D_h)),
            w_spec((D_h, D_h)),
            w_spec((D_out, D_h)),
            w_spec((1, D_out)),
        ],
        out_specs=pl.BlockSpec((1, D_out, N), lambda b: (b, 0, 0)),
        out_shape=jax.ShapeDtypeStruct((B, D_out, N), jnp.float32),
        compiler_params=pltpu.CompilerParams(
            dimension_semantics=("parallel",)),
    )(h_0, W_in, W_h1, W_h2, W_out, W_cls_t, b2)

    return jnp.transpose(y_t, (0, 2, 1))


# speedup vs baseline: 1.2337x; 1.0022x over previous
import jax
import jax.numpy as jnp
from jax.experimental import pallas as pl
from jax.experimental.pallas import tpu as pltpu


def _dot(a, b):
    return jnp.dot(a.astype(jnp.bfloat16), b.astype(jnp.bfloat16),
                   preferred_element_type=jnp.float32)


def _fused_mlp_kernel(x_ref, w_in_ref, w_h1_ref, w_h2_ref, w_out_ref,
                      w_cls_t_ref, b_cls_ref, out_ref):
    x = x_ref[0]
    h = jax.nn.relu(_dot(x, w_in_ref[...]).astype(jnp.bfloat16))
    h = jax.nn.relu(_dot(h, w_h1_ref[...]).astype(jnp.bfloat16))
    h = jax.nn.relu(_dot(h, w_h2_ref[...]).astype(jnp.bfloat16))
    w_comb_t = jax.lax.dot_general(
        w_cls_t_ref[...].astype(jnp.bfloat16),
        w_out_ref[...].astype(jnp.bfloat16),
        dimension_numbers=(((1,), (1,)), ((), ())),
        preferred_element_type=jnp.float32)
    y_t = jax.lax.dot_general(
        w_comb_t.astype(jnp.bfloat16), h,
        dimension_numbers=(((1,), (1,)), ((), ())),
        preferred_element_type=jnp.float32)
    b = jax.lax.broadcast_in_dim(b_cls_ref[0], y_t.shape, (0,))
    out_ref[0] = y_t + b


def kernel(h_0, W_in, W_h1, W_h2, W_out, W_cls, b_cls):
    B, N, D_in = h_0.shape
    D_h = W_in.shape[1]
    D_out = W_cls.shape[1]
    W_cls_t = W_cls.T
    b2 = b_cls.reshape(1, D_out)

    grid = (B,)

    def w_spec(shape):
        return pl.BlockSpec(shape, lambda b: (0, 0))

    y_t = pl.pallas_call(
        _fused_mlp_kernel,
        grid=grid,
        in_specs=[
            pl.BlockSpec((1, N, D_in), lambda b: (b, 0, 0)),
            w_spec((D_in, D_h)),
            w_spec((D_h, D_h)),
            w_spec((D_h, D_h)),
            w_spec((D_h, D_h)),
            w_spec((D_out, D_h)),
            w_spec((1, D_out)),
        ],
        out_specs=pl.BlockSpec((1, D_out, N), lambda b: (b, 0, 0)),
        out_shape=jax.ShapeDtypeStruct((B, D_out, N), jnp.float32),
        compiler_params=pltpu.CompilerParams(
            dimension_semantics=("parallel",)),
    )(h_0, W_in, W_h1, W_h2, W_out, W_cls_t, b2)

    return jnp.transpose(y_t, (0, 2, 1))
